# baseline (speedup 1.0000x reference)
import functools

import jax
import jax.numpy as jnp
from jax import lax
from jax.experimental import pallas as pl
from jax.experimental.pallas import tpu as pltpu
from jax.experimental.pallas import tpu_sc as plsc

_NC = 2
_NS = 16
_NW = _NC * _NS
_B = 16384
_K = 32
_TBL = 1000000
_BPW = _B // _NW
_CH = 128
_NCH = _BPW // _CH


def _body(xt_hbm, wt_hbm, ht_hbm, out_hbm, idx_u, idx_v, z, sem):
    wid = lax.axis_index("s") * _NC + lax.axis_index("c")
    base = wid * _BPW
    pltpu.sync_copy(xt_hbm.at[0, pl.ds(base, _BPW)], idx_u)
    pltpu.sync_copy(xt_hbm.at[1, pl.ds(base, _BPW)], idx_v)

    def issue(c, carry):
        for j in range(_NCH):
            s = pl.ds(j * _CH, _CH)
            pltpu.async_copy(
                wt_hbm.at[c].at[idx_u.at[s]], z.at[c, s], sem)
            pltpu.async_copy(
                ht_hbm.at[c].at[idx_v.at[s]], z.at[_K + c, s], sem)
        return carry

    lax.fori_loop(0, _K, issue, 0)

    def drain(i, carry):
        pltpu.make_async_copy(
            wt_hbm.at[0].at[idx_u.at[pl.ds(0, _CH)]],
            z.at[0, pl.ds(0, _CH)], sem).wait()
        return carry

    lax.fori_loop(0, 2 * _K * _NCH, drain, 0)

    pltpu.sync_copy(z, out_hbm.at[:, pl.ds(base, _BPW)])


@functools.partial(
    pl.kernel,
    mesh=plsc.VectorSubcoreMesh(core_axis_name="c", subcore_axis_name="s"),
    compiler_params=pltpu.CompilerParams(
        needs_layout_passes=False, use_tc_tiling_on_sc=False
    ),
    out_type=jax.ShapeDtypeStruct((2 * _K, _B), jnp.float32),
    scratch_types=[
        pltpu.VMEM((_BPW,), jnp.int32),
        pltpu.VMEM((_BPW,), jnp.int32),
        pltpu.VMEM((2 * _K, _BPW), jnp.float32),
        pltpu.SemaphoreType.DMA,
    ],
)
def _lookup(xt_hbm, wt_hbm, ht_hbm, out_hbm, idx_u, idx_v, z, sem):
    _body(xt_hbm, wt_hbm, ht_hbm, out_hbm, idx_u, idx_v, z, sem)


def kernel(x, W, H):
    out_t = _lookup(x.T, W.T, H.T)
    return out_t.T


# speedup vs baseline: 5.6403x; 5.6403x over previous
import functools

import jax
import jax.numpy as jnp
from jax import lax
from jax.experimental import pallas as pl
from jax.experimental.pallas import tpu as pltpu
from jax.experimental.pallas import tpu_sc as plsc

_NC = 2
_NS = 16
_NW = _NC * _NS
_B = 16384
_K = 32
_L = 16
_BPW = _B // _NW
_CH = 128
_NCH = _BPW // _CH
_GPC = _CH // _L


def _body(x_hbm, w_hbm, h_hbm, out_hbm, xy, idx_u, idx_v, dst_u, dst_v,
          rows_u, rows_v, sem):
    wid = lax.axis_index("s") * _NC + lax.axis_index("c")
    base = wid * _BPW
    pltpu.sync_copy(x_hbm.at[pl.ds(base * 2, _BPW * 2)], xy)
    lane = lax.iota(jnp.int32, _L)
    for j in range(_NCH):
        for g in range(_GPC):
            p = j * _CH + g * _L
            src = (lane + p) * 2
            idx_u[j, pl.ds(g * _L, _L)] = plsc.load_gather(xy, [src])
            idx_v[j, pl.ds(g * _L, _L)] = plsc.load_gather(xy, [src + 1])
            drow = (base + p + lane) * 2
            dst_u[j, pl.ds(g * _L, _L)] = drow
            dst_v[j, pl.ds(g * _L, _L)] = drow + 1
    copies = []
    for j in range(_NCH):
        s = pl.ds(j * _CH, _CH)
        copies.append(pltpu.async_copy(w_hbm.at[idx_u.at[j]], rows_u.at[s], sem))
        copies.append(pltpu.async_copy(h_hbm.at[idx_v.at[j]], rows_v.at[s], sem))
    for c in copies:
        c.wait()
    copies = []
    for j in range(_NCH):
        s = pl.ds(j * _CH, _CH)
        copies.append(pltpu.async_copy(rows_u.at[s], out_hbm.at[dst_u.at[j]], sem))
        copies.append(pltpu.async_copy(rows_v.at[s], out_hbm.at[dst_v.at[j]], sem))
    for c in copies:
        c.wait()


@functools.partial(
    pl.kernel,
    mesh=plsc.VectorSubcoreMesh(core_axis_name="c", subcore_axis_name="s"),
    compiler_params=pltpu.CompilerParams(
        needs_layout_passes=False, use_tc_tiling_on_sc=False
    ),
    out_type=jax.ShapeDtypeStruct((2 * _B, _K), jnp.float32),
    scratch_types=[
        pltpu.VMEM((2 * _BPW,), jnp.int32),
        pltpu.VMEM((_NCH, _CH), jnp.int32),
        pltpu.VMEM((_NCH, _CH), jnp.int32),
        pltpu.VMEM((_NCH, _CH), jnp.int32),
        pltpu.VMEM((_NCH, _CH), jnp.int32),
        pltpu.VMEM((_BPW, _K), jnp.float32),
        pltpu.VMEM((_BPW, _K), jnp.float32),
        pltpu.SemaphoreType.DMA,
    ],
)
def _lookup(x_hbm, w_hbm, h_hbm, out_hbm, xy, idx_u, idx_v, dst_u, dst_v,
            rows_u, rows_v, sem):
    _body(x_hbm, w_hbm, h_hbm, out_hbm, xy, idx_u, idx_v, dst_u, dst_v,
          rows_u, rows_v, sem)


def kernel(x, W, H):
    out = _lookup(x.reshape(-1), W, H)
    return out.reshape(_B, 2 * _K)


# speedup vs baseline: 23.5774x; 4.1802x over previous
import functools

import jax
import jax.numpy as jnp
from jax import lax
from jax.experimental import pallas as pl
from jax.experimental.pallas import tpu as pltpu
from jax.experimental.pallas import tpu_sc as plsc

_NC = 2
_NS = 16
_NW = _NC * _NS
_B = 16384
_K = 32
_BPW = _B // _NW
_NB = 4
_NG = _BPW // 16


def _extract(strips, slot, col, zrow, zoff, z):
    rows_lo = lax.iota(jnp.int32, 16)
    cols = jnp.full((16,), col, jnp.int32)
    z[zrow, pl.ds(zoff, 16)] = plsc.load_gather(strips.at[slot], [rows_lo, cols])
    z[zrow, pl.ds(zoff + 16, 16)] = plsc.load_gather(
        strips.at[slot], [rows_lo + 16, cols])


def _body(xt, wt, ht, out, idx_u, idx_v, sw, sh, z, sem):
    wid = lax.axis_index("s") * _NC + lax.axis_index("c")
    base = wid * _BPW
    pltpu.sync_copy(xt.at[0, pl.ds(base, _BPW)], idx_u)
    pltpu.sync_copy(xt.at[1, pl.ds(base, _BPW)], idx_v)

    def fetch(i, slot, tbl, bufs):
        t = (i >> 7) << 7
        pltpu.async_copy(tbl.at[:, pl.ds(pl.multiple_of(t, 128), 128)],
                         bufs.at[slot], sem)

    def drain_one(tbl, bufs, slot):
        pltpu.make_async_copy(tbl.at[:, pl.ds(0, 128)], bufs.at[slot],
                              sem).wait()

    def group(g, carry):
        iv_u = idx_u[pl.ds(g * 16, 16)]
        iv_v = idx_v[pl.ds(g * 16, 16)]
        gp = jnp.maximum(g - 1, 0)
        pv_u = idx_u[pl.ds(gp * 16, 16)]
        pv_v = idx_v[pl.ds(gp * 16, 16)]
        for l in range(16):
            slot = l % _NB
            if l < _NB:
                @pl.when(g > 0)
                def _():
                    drain_one(wt, sw, slot)
                    drain_one(ht, sh, slot)
                    lag = gp * 16 + l + 12
                    _extract(sw, slot, pv_u[l + 12] & 127, lag, 0, z)
                    _extract(sh, slot, pv_v[l + 12] & 127, lag, _K, z)
            else:
                drain_one(wt, sw, slot)
                drain_one(ht, sh, slot)
                lag = g * 16 + l - _NB
                _extract(sw, slot, iv_u[l - _NB] & 127, lag, 0, z)
                _extract(sh, slot, iv_v[l - _NB] & 127, lag, _K, z)
            fetch(iv_u[l], slot, wt, sw)
            fetch(iv_v[l], slot, ht, sh)
        return carry

    lax.fori_loop(0, _NG, group, 0)

    iv_u = idx_u[pl.ds((_NG - 1) * 16, 16)]
    iv_v = idx_v[pl.ds((_NG - 1) * 16, 16)]
    for l in range(16 - _NB, 16):
        slot = l % _NB
        drain_one(wt, sw, slot)
        drain_one(ht, sh, slot)
        e = (_NG - 1) * 16 + l
        _extract(sw, slot, iv_u[l] & 127, e, 0, z)
        _extract(sh, slot, iv_v[l] & 127, e, _K, z)

    pltpu.sync_copy(z, out.at[pl.ds(base, _BPW), :])


@functools.partial(
    pl.kernel,
    mesh=plsc.VectorSubcoreMesh(core_axis_name="c", subcore_axis_name="s"),
    compiler_params=pltpu.CompilerParams(
        needs_layout_passes=False, use_tc_tiling_on_sc=True
    ),
    out_type=jax.ShapeDtypeStruct((_B, 2 * _K), jnp.float32),
    scratch_types=[
        pltpu.VMEM((_BPW,), jnp.int32),
        pltpu.VMEM((_BPW,), jnp.int32),
        pltpu.VMEM((_NB, _K, 128), jnp.float32),
        pltpu.VMEM((_NB, _K, 128), jnp.float32),
        pltpu.VMEM((_BPW, 2 * _K), jnp.float32),
        pltpu.SemaphoreType.DMA,
    ],
)
def _lookup(xt, wt, ht, out, idx_u, idx_v, sw, sh, z, sem):
    _body(xt, wt, ht, out, idx_u, idx_v, sw, sh, z, sem)


def kernel(x, W, H):
    return _lookup(x.T, W.T, H.T)
